# baseline (speedup 1.0000x reference)
import functools

import jax
import jax.numpy as jnp
from jax import lax
from jax.experimental import pallas as pl
from jax.experimental.pallas import tpu as pltpu
from jax.experimental.pallas import tpu_sc as plsc

N = 10000
E = 160000
DF = 256
H = 512
O = 256
NC, NS = 2, 16
NW = NC * NS
EPT = E // NW
NP_ = 10240
RPT = NP_ // NS

_MESH = dict(core_axis_name="c", subcore_axis_name="s")


def _deg_body(dst3, ones_hbm, zeros_hbm, out, dstb, onesb, zb, acc):
    c = lax.axis_index("c")
    s = lax.axis_index("s")
    wid = c * NS + s
    pltpu.sync_copy(dst3.at[wid], dstb)
    pltpu.sync_copy(ones_hbm, onesb)
    pltpu.sync_copy(zeros_hbm, zb)
    for j in range(5):
        pltpu.sync_copy(zb, acc.at[pl.ds(s * RPT + j * 128, 128)])
    plsc.subcore_barrier()

    def body(i, carry):
        pltpu.sync_copy(onesb, acc.at[dstb.at[i]], add=True)
        return carry

    lax.fori_loop(0, 50, body, 0)
    plsc.subcore_barrier()
    pltpu.sync_copy(acc.at[pl.ds(s * RPT, RPT)], out.at[c, pl.ds(s * RPT, RPT)])


def _make_deg():
    return functools.partial(
        pl.kernel,
        mesh=plsc.VectorSubcoreMesh(**_MESH),
        out_type=jax.ShapeDtypeStruct((NC, NP_, 128), jnp.float32),
        scratch_types=[
            pltpu.VMEM((50, 100), jnp.int32),
            pltpu.VMEM((100, 128), jnp.float32),
            pltpu.VMEM((128, 128), jnp.float32),
            pltpu.VMEM_SHARED((NP_, 128), jnp.float32),
        ],
    )(_deg_body)


def _asm_body(idx0, t_hbm, ph_hbm, qh_hbm, ib, ibq,
              big0, big1, big2, big3, big4, big5, big6, big7,
              sg0, sg1, sg2, sg3, sg4, sg5, sg6, sg7,
              sa0, sa1, sa2, sa3, sa4, sa5, sa6, sa7,
              sb0, sb1, sb2, sb3, sb4, sb5, sb6, sb7):
    c = lax.axis_index("c")
    s = lax.axis_index("s")
    wid = c * NS + s
    base = wid * EPT
    nb = EPT // 40
    bigs = (big0, big1, big2, big3, big4, big5, big6, big7)
    sgs = (sg0, sg1, sg2, sg3, sg4, sg5, sg6, sg7)
    sas = (sa0, sa1, sa2, sa3, sa4, sa5, sa6, sa7)
    sbs = (sb0, sb1, sb2, sb3, sb4, sb5, sb6, sb7)
    pltpu.sync_copy(idx0.at[wid], ib)

    def set_idx(k, b):
        for v in range(5):
            sl = pl.ds(v * 16, 16)
            ibq[k, sl] = ib[b, sl] + (2 * (k % 2) * N)

    def gather(k):
        pltpu.async_copy(t_hbm.at[ibq.at[k]], bigs[k], sgs[k])

    def gwait(k):
        pltpu.make_async_copy(t_hbm.at[ibq.at[k]], bigs[k], sgs[k]).wait()

    def wwait(k):
        h = k % 2
        pltpu.make_async_copy(
            bigs[k].at[pl.ds(0, 40)],
            ph_hbm.at[h, pl.ds(base, 40)], sas[k]).wait()
        pltpu.make_async_copy(
            bigs[k].at[pl.ds(40, 40)],
            qh_hbm.at[h, pl.ds(base, 40)], sbs[k]).wait()

    for k in range(4):
        set_idx(k, k // 2)
        gather(k)

    def outer(t, carry):
        for k in range(8):
            h = k % 2
            b = jnp.minimum((8 * t + k) // 2, nb - 1)
            k4 = (k + 4) % 8
            b4 = jnp.minimum((8 * t + k + 4) // 2, nb - 1)
            gwait(k)
            pltpu.async_copy(
                bigs[k].at[pl.ds(0, 40)],
                ph_hbm.at[h, pl.ds(base + b * 40, 40)], sas[k])
            pltpu.async_copy(
                bigs[k].at[pl.ds(40, 40)],
                qh_hbm.at[h, pl.ds(base + b * 40, 40)], sbs[k])
            if k < 4:
                @pl.when(t > 0)
                def _():
                    wwait(k4)
            else:
                wwait(k4)
            set_idx(k4, b4)
            gather(k4)
        return carry

    lax.fori_loop(0, (2 * nb + 7) // 8, outer, 0)
    for k in range(4):
        gwait(k)
    for k in range(4, 8):
        wwait(k)


def _make_asm():
    sds = jax.ShapeDtypeStruct
    return functools.partial(
        pl.kernel,
        mesh=plsc.VectorSubcoreMesh(**_MESH),
        out_type=[sds((2, E, 128), jnp.int32), sds((2, E, 128), jnp.int32)],
        scratch_types=(
            [pltpu.VMEM((EPT // 40, 80), jnp.int32),
             pltpu.VMEM((8, 80), jnp.int32)]
            + [pltpu.VMEM((80, 128), jnp.int32)] * 8
            + [pltpu.SemaphoreType.DMA] * 24
        ),
    )(_asm_body)


def _make_scatter(nchunks):
    def body(*args):
        src3, dst3, zeros_hbm = args[0], args[1], args[2]
        tabs = args[3:3 + nchunks]
        out = args[3 + nchunks]
        srcb, dstb, rows0, rows1, acc, sem0, sem1, ssem0, ssem1 = args[4 + nchunks:]
        c = lax.axis_index("c")
        s = lax.axis_index("s")
        wid = c * NS + s
        pltpu.sync_copy(src3.at[wid], srcb)
        pltpu.sync_copy(dst3.at[wid], dstb)
        for ch in range(nchunks):
            for j in range(5):
                pltpu.sync_copy(zeros_hbm, acc.at[pl.ds(s * RPT + j * 128, 128)])
            plsc.subcore_barrier()

            _tab = tabs[ch]
            pltpu.async_copy(_tab.at[srcb.at[0]], rows0, sem0)

            def ebody(g, carry):
                i0 = 2 * g
                i2 = jnp.minimum(i0 + 2, 49)
                pltpu.async_copy(_tab.at[srcb.at[i0 + 1]], rows1, sem1)
                pltpu.make_async_copy(_tab.at[srcb.at[0]], rows0, sem0).wait()
                w0 = pltpu.async_copy(rows0, acc.at[dstb.at[i0]], add=True,
                                      sem=ssem0)
                pltpu.make_async_copy(_tab.at[srcb.at[0]], rows1, sem1).wait()
                w1 = pltpu.async_copy(rows1, acc.at[dstb.at[i0 + 1]], add=True,
                                      sem=ssem1)
                w0.wait()
                pltpu.async_copy(_tab.at[srcb.at[i2]], rows0, sem0)
                w1.wait()
                return carry

            lax.fori_loop(0, 25, ebody, 0)
            pltpu.make_async_copy(_tab.at[srcb.at[0]], rows0, sem0).wait()
            plsc.subcore_barrier()
            pltpu.sync_copy(
                acc.at[pl.ds(s * RPT, RPT)],
                out.at[c, pl.ds(s * RPT, RPT), pl.ds(ch * 128, 128)])
            plsc.subcore_barrier()

    sds = jax.ShapeDtypeStruct
    return functools.partial(
        pl.kernel,
        mesh=plsc.VectorSubcoreMesh(**_MESH),
        out_type=sds((NC, NP_, nchunks * 128), jnp.float32),
        scratch_types=[
            pltpu.VMEM((50, 100), jnp.int32),
            pltpu.VMEM((50, 100), jnp.int32),
            pltpu.VMEM((100, 128), jnp.float32),
            pltpu.VMEM((100, 128), jnp.float32),
            pltpu.VMEM_SHARED((NP_, 128), jnp.float32),
            pltpu.SemaphoreType.DMA,
            pltpu.SemaphoreType.DMA,
            pltpu.SemaphoreType.DMA,
            pltpu.SemaphoreType.DMA,
        ],
    )(body)


def _pq_kernel(x, w_pq):
    bm = 2000

    def body(x_ref, w_ref, t_ref):
        pq = jnp.dot(x_ref[...], w_ref[...], preferred_element_type=jnp.float32)
        pqh = pq.astype(jnp.bfloat16)
        for h in range(2):
            t_ref[2 * h] = pqh[:, 256 * h:256 * (h + 1)]
            t_ref[2 * h + 1] = pqh[:, 512 + 256 * h:512 + 256 * (h + 1)]

    return pl.pallas_call(
        body,
        grid=(N // bm,),
        in_specs=[pl.BlockSpec((bm, DF), lambda i: (i, 0)),
                  pl.BlockSpec((DF, 2 * H), lambda i: (0, 0))],
        out_specs=pl.BlockSpec((4, bm, 256), lambda i: (0, i, 0)),
        out_shape=jax.ShapeDtypeStruct((4, N, 256), jnp.bfloat16),
    )(x, w_pq)


def _norm_kernel(deg0, deg1):
    bm = 2000

    def body(a_ref, b_ref, cdeg_ref, dinv_ref):
        d = 1.0 + a_ref[:, :16] + b_ref[:, :16]
        cdeg_ref[...] = 1.0 / d
        dinv_ref[...] = lax.rsqrt(d)

    return pl.pallas_call(
        body,
        grid=(N // bm,),
        in_specs=[pl.BlockSpec((bm, 128), lambda i: (i, 0)),
                  pl.BlockSpec((bm, 128), lambda i: (i, 0))],
        out_specs=[pl.BlockSpec((bm, 16), lambda i: (i, 0)),
                   pl.BlockSpec((bm, 16), lambda i: (i, 0))],
        out_shape=[jax.ShapeDtypeStruct((N, 16), jnp.float32),
                   jax.ShapeDtypeStruct((N, 16), jnp.float32)],
    )(deg0, deg1)


def _z1_blk(ph_ref, qh_ref, ea_ref, wc_ref):
    z1 = jnp.concatenate(
        [ph_ref[0].astype(jnp.float32) + qh_ref[0].astype(jnp.float32),
         ph_ref[1].astype(jnp.float32) + qh_ref[1].astype(jnp.float32)],
        axis=1)
    return z1 + jnp.dot(ea_ref[...].astype(jnp.bfloat16),
                        wc_ref[...].astype(jnp.bfloat16),
                        preferred_element_type=jnp.float32)


def _u1_kernel(ph, qh, ea, w1c, dinvw):
    bm = 2000

    def body(ph_ref, qh_ref, ea_ref, wc_ref, dv_ref, *out_refs):
        u = dv_ref[:, :1] * _z1_blk(ph_ref, qh_ref, ea_ref, wc_ref)
        for ch, o_ref in enumerate(out_refs):
            o_ref[...] = u[:, ch * 128:(ch + 1) * 128]

    return pl.pallas_call(
        body,
        grid=(N // bm,),
        in_specs=[pl.BlockSpec((2, bm, 256), lambda i: (0, i, 0)),
                  pl.BlockSpec((2, bm, 256), lambda i: (0, i, 0)),
                  pl.BlockSpec((bm, 16), lambda i: (i, 0)),
                  pl.BlockSpec((16, H), lambda i: (0, 0)),
                  pl.BlockSpec((bm, 16), lambda i: (i, 0))],
        out_specs=[pl.BlockSpec((bm, 128), lambda i: (i, 0))] * (H // 128),
        out_shape=[jax.ShapeDtypeStruct((N, 128), jnp.float32)] * (H // 128),
    )(ph, qh, ea, w1c, dinvw)


def _layer2_kernel(ph, qh, ea, w1c, agg1, cdegw, dinvw, w2, b1r, b2r):
    bm = 2000
    nhead = N // bm

    def body(ph_ref, qh_ref, ea_ref, wc_ref, agg_ref, cd_ref, dv_ref,
             w_ref, b1_ref, b2_ref, acc_ref, t2_ref, u0_ref, u1_ref):
        i = pl.program_id(0)
        z1 = _z1_blk(ph_ref, qh_ref, ea_ref, wc_ref)
        b1v = b1_ref[...]
        cd = cd_ref[:, :1]
        dv = dv_ref[:, :1]
        agg = agg_ref[0] + agg_ref[1]
        fixed = cd * z1 + dv * agg + b1v
        plain = z1 + b1v
        a = jax.nn.relu(jnp.where(i < nhead, fixed, plain))
        z2 = jnp.dot(a.astype(jnp.bfloat16), w_ref[...].astype(jnp.bfloat16),
                     preferred_element_type=jnp.float32)

        @pl.when(i < nhead)
        def _():
            t2_ref[...] = z2
            u2 = dv * z2
            u0_ref[...] = u2[:, :128]
            u1_ref[...] = u2[:, 128:]

        @pl.when(i == 0)
        def _():
            acc_ref[...] = jnp.zeros_like(acc_ref)

        @pl.when(i >= nhead)
        def _():
            acc_ref[...] += jnp.sum(jax.nn.relu(z2 + b2_ref[...]),
                                    axis=0, keepdims=True)

    head = lambda i: (jnp.minimum(i, nhead - 1), 0)
    return pl.pallas_call(
        body,
        grid=(E // bm,),
        in_specs=[
            pl.BlockSpec((2, bm, 256), lambda i: (0, i, 0)),
            pl.BlockSpec((2, bm, 256), lambda i: (0, i, 0)),
            pl.BlockSpec((bm, 16), lambda i: (i, 0)),
            pl.BlockSpec((16, H), lambda i: (0, 0)),
            pl.BlockSpec((NC, bm, H), lambda i: (0, jnp.minimum(i, nhead - 1), 0)),
            pl.BlockSpec((bm, 16), head),
            pl.BlockSpec((bm, 16), head),
            pl.BlockSpec((H, O), lambda i: (0, 0)),
            pl.BlockSpec((1, H), lambda i: (0, 0)),
            pl.BlockSpec((1, O), lambda i: (0, 0)),
        ],
        out_specs=[
            pl.BlockSpec((1, O), lambda i: (0, 0)),
            pl.BlockSpec((bm, O), head),
            pl.BlockSpec((bm, 128), head),
            pl.BlockSpec((bm, 128), head),
        ],
        out_shape=[
            jax.ShapeDtypeStruct((1, O), jnp.float32),
            jax.ShapeDtypeStruct((N, O), jnp.float32),
            jax.ShapeDtypeStruct((N, 128), jnp.float32),
            jax.ShapeDtypeStruct((N, 128), jnp.float32),
        ],
    )(ph, qh, ea, w1c, agg1, cdegw, dinvw, w2, b1r, b2r)


def _final_kernel(partial, t2, agg2, cdegw, dinvw, b2r, fcw_t, fcb_r):
    bm = 2000
    nblk = N // bm

    def body(part_ref, t2_ref, agg_ref, cd_ref, dv_ref, b2_ref,
             fw_ref, fb_ref, out_ref, s_ref):
        i = pl.program_id(0)

        @pl.when(i == 0)
        def _():
            s_ref[...] = part_ref[...]

        rows = jax.nn.relu(cd_ref[:, :1] * t2_ref[...]
                           + dv_ref[:, :1] * (agg_ref[0] + agg_ref[1])
                           + b2_ref[...])
        s_ref[...] += jnp.sum(rows, axis=0, keepdims=True)

        @pl.when(i == nblk - 1)
        def _():
            out_ref[...] = jnp.dot(s_ref[...], fw_ref[...],
                                   preferred_element_type=jnp.float32) + fb_ref[...]

    return pl.pallas_call(
        body,
        grid=(nblk,),
        in_specs=[
            pl.BlockSpec((1, O), lambda i: (0, 0)),
            pl.BlockSpec((bm, O), lambda i: (i, 0)),
            pl.BlockSpec((NC, bm, O), lambda i: (0, i, 0)),
            pl.BlockSpec((bm, 16), lambda i: (i, 0)),
            pl.BlockSpec((bm, 16), lambda i: (i, 0)),
            pl.BlockSpec((1, O), lambda i: (0, 0)),
            pl.BlockSpec((O, O), lambda i: (0, 0)),
            pl.BlockSpec((1, O), lambda i: (0, 0)),
        ],
        out_specs=pl.BlockSpec((1, O), lambda i: (0, 0)),
        out_shape=jax.ShapeDtypeStruct((1, O), jnp.float32),
        scratch_shapes=[pltpu.VMEM((1, O), jnp.float32)],
    )(partial, t2, agg2, cdegw, dinvw, b2r, fcw_t, fcb_r)


def kernel(x, edge_index, edge_attr, W1, b1, W2, b2, fcW, fcb):
    src = edge_index[0]
    dst = edge_index[1]
    src_g = src.reshape(NW, EPT // 40, 40)
    dstN_g = (dst + N).reshape(NW, EPT // 40, 40)
    idx0 = jnp.concatenate([src_g, dstN_g], axis=2)
    src_s = src.reshape(NW, 50, 100)
    dst_s = dst.reshape(NW, 50, 100)

    ones128 = jnp.ones((100, 128), jnp.float32)
    zeros128 = jnp.zeros((128, 128), jnp.float32)

    w_pq = jnp.concatenate([W1[:DF], W1[DF:2 * DF]], axis=1)
    w1c = W1[2 * DF:]
    b1r = b1.reshape(1, H)
    b2r = b2.reshape(1, O)
    fcw_t = fcW.T
    fcb_r = fcb.reshape(1, O)

    degw = _make_deg()(dst_s, ones128, zeros128)
    cdegw, dinvw = _norm_kernel(degw[0], degw[1])
    t_bf = _pq_kernel(x, w_pq).reshape(4 * N, 128, 2)
    t_tab = lax.bitcast_convert_type(t_bf, jnp.int32)

    ph32, qh32 = _make_asm()(idx0, t_tab)
    ph = lax.bitcast_convert_type(ph32, jnp.bfloat16).reshape(2, E, 256)
    qh = lax.bitcast_convert_type(qh32, jnp.bfloat16).reshape(2, E, 256)

    u1 = _u1_kernel(ph, qh, edge_attr, w1c, dinvw)
    agg1 = _make_scatter(4)(src_s, dst_s, zeros128, *u1)

    partial, t2, u2c0, u2c1 = _layer2_kernel(
        ph, qh, edge_attr, w1c, agg1, cdegw, dinvw, W2, b1r, b2r)

    agg2 = _make_scatter(2)(src_s, dst_s, zeros128, u2c0, u2c1)

    out = _final_kernel(partial, t2, agg2, cdegw, dinvw, b2r, fcw_t, fcb_r)
    return out.reshape(O)


# speedup vs baseline: 2.4515x; 2.4515x over previous
import functools

import jax
import jax.numpy as jnp
from jax import lax
from jax.experimental import pallas as pl
from jax.experimental.pallas import tpu as pltpu
from jax.experimental.pallas import tpu_sc as plsc

N = 10000
E = 160000
DF = 256
H = 512
O = 256
NC, NS = 2, 16
NW = NC * NS
EPT = E // NW
NP_ = 10240
RPT = NP_ // NS

_MESH = dict(core_axis_name="c", subcore_axis_name="s")


def _deg_body(dst3, ones_hbm, zeros_hbm, out, dstb, onesb, zb, acc):
    c = lax.axis_index("c")
    s = lax.axis_index("s")
    wid = c * NS + s
    pltpu.sync_copy(dst3.at[wid], dstb)
    pltpu.sync_copy(ones_hbm, onesb)
    pltpu.sync_copy(zeros_hbm, zb)
    for j in range(5):
        pltpu.sync_copy(zb, acc.at[pl.ds(s * RPT + j * 128, 128)])
    plsc.subcore_barrier()

    def body(i, carry):
        pltpu.sync_copy(onesb, acc.at[dstb.at[i]], add=True)
        return carry

    lax.fori_loop(0, 50, body, 0)
    plsc.subcore_barrier()
    pltpu.sync_copy(acc.at[pl.ds(s * RPT, RPT)], out.at[c, pl.ds(s * RPT, RPT)])


def _make_deg():
    return functools.partial(
        pl.kernel,
        mesh=plsc.VectorSubcoreMesh(**_MESH),
        out_type=jax.ShapeDtypeStruct((NC, NP_, 128), jnp.float32),
        scratch_types=[
            pltpu.VMEM((50, 100), jnp.int32),
            pltpu.VMEM((100, 128), jnp.float32),
            pltpu.VMEM((128, 128), jnp.float32),
            pltpu.VMEM_SHARED((NP_, 128), jnp.float32),
        ],
    )(_deg_body)


def _asm_body(idx0, t_hbm, ps_hbm, ib, ibq,
              big0, big1, big2, big3, eb0, eb1, eb2, eb3,
              sg0, sg1, sg2, sg3, sw0, sw1, sw2, sw3):
    c = lax.axis_index("c")
    s = lax.axis_index("s")
    wid = c * NS + s
    base = wid * EPT
    nb = EPT // 40
    bigs = (big0, big1, big2, big3)
    ebs = (eb0, eb1, eb2, eb3)
    sgs = (sg0, sg1, sg2, sg3)
    sws = (sw0, sw1, sw2, sw3)
    pltpu.sync_copy(idx0.at[wid], ib)

    def set_idx(g, q):
        for v in range(5):
            sl = pl.ds(v * 16, 16)
            ibq[q, sl] = ib[g, sl] + (2 * q * N)

    def gather(q):
        pltpu.async_copy(t_hbm.at[ibq.at[q]], bigs[q], sgs[q])

    def gwait(q):
        pltpu.make_async_copy(t_hbm.at[ibq.at[q]], bigs[q], sgs[q]).wait()

    def combine(q):
        big, eb = bigs[q], ebs[q]

        def inner_j(j, cj):
            for k in range(8):
                sl = pl.ds(k * 16, 16)
                eb[j, sl] = big[j, sl] + big[40 + j, sl]
            return cj
        lax.fori_loop(0, 40, inner_j, 0)

    for q in range(4):
        set_idx(0, q)
        gather(q)

    def outer(g, carry):
        gn = jnp.minimum(g + 1, nb - 1)
        for q in range(4):
            gwait(q)
            combine(q)
            w = pltpu.async_copy(
                ebs[q], ps_hbm.at[q, pl.ds(base + g * 40, 40)], sws[q])
            set_idx(gn, q)
            gather(q)
            w.wait()
        return carry

    lax.fori_loop(0, nb, outer, 0)
    for q in range(4):
        gwait(q)


def _make_asm():
    return functools.partial(
        pl.kernel,
        mesh=plsc.VectorSubcoreMesh(**_MESH),
        out_type=jax.ShapeDtypeStruct((4, E, 128), jnp.float32),
        scratch_types=(
            [pltpu.VMEM((EPT // 40, 80), jnp.int32),
             pltpu.VMEM((4, 80), jnp.int32)]
            + [pltpu.VMEM((80, 128), jnp.float32)] * 4
            + [pltpu.VMEM((40, 128), jnp.float32)] * 4
            + [pltpu.SemaphoreType.DMA] * 8
        ),
    )(_asm_body)


def _make_scatter(nchunks):
    def body(*args):
        src3, dst3, zeros_hbm = args[0], args[1], args[2]
        tabs = args[3:3 + nchunks]
        out = args[3 + nchunks]
        srcb, dstb, rows0, rows1, acc, sem0, sem1, ssem0, ssem1 = args[4 + nchunks:]
        c = lax.axis_index("c")
        s = lax.axis_index("s")
        wid = c * NS + s
        pltpu.sync_copy(src3.at[wid], srcb)
        pltpu.sync_copy(dst3.at[wid], dstb)
        for ch in range(nchunks):
            for j in range(5):
                pltpu.sync_copy(zeros_hbm, acc.at[pl.ds(s * RPT + j * 128, 128)])
            plsc.subcore_barrier()

            _tab = tabs[ch]
            pltpu.async_copy(_tab.at[srcb.at[0]], rows0, sem0)

            def ebody(g, carry):
                i0 = 2 * g
                i2 = jnp.minimum(i0 + 2, 49)
                pltpu.async_copy(_tab.at[srcb.at[i0 + 1]], rows1, sem1)
                pltpu.make_async_copy(_tab.at[srcb.at[0]], rows0, sem0).wait()
                w0 = pltpu.async_copy(rows0, acc.at[dstb.at[i0]], add=True,
                                      sem=ssem0)
                pltpu.make_async_copy(_tab.at[srcb.at[0]], rows1, sem1).wait()
                w1 = pltpu.async_copy(rows1, acc.at[dstb.at[i0 + 1]], add=True,
                                      sem=ssem1)
                w0.wait()
                pltpu.async_copy(_tab.at[srcb.at[i2]], rows0, sem0)
                w1.wait()
                return carry

            lax.fori_loop(0, 25, ebody, 0)
            pltpu.make_async_copy(_tab.at[srcb.at[0]], rows0, sem0).wait()
            plsc.subcore_barrier()
            pltpu.sync_copy(
                acc.at[pl.ds(s * RPT, RPT)],
                out.at[c, pl.ds(s * RPT, RPT), pl.ds(ch * 128, 128)])
            plsc.subcore_barrier()

    sds = jax.ShapeDtypeStruct
    return functools.partial(
        pl.kernel,
        mesh=plsc.VectorSubcoreMesh(**_MESH),
        out_type=sds((NC, NP_, nchunks * 128), jnp.float32),
        scratch_types=[
            pltpu.VMEM((50, 100), jnp.int32),
            pltpu.VMEM((50, 100), jnp.int32),
            pltpu.VMEM((100, 128), jnp.float32),
            pltpu.VMEM((100, 128), jnp.float32),
            pltpu.VMEM_SHARED((NP_, 128), jnp.float32),
            pltpu.SemaphoreType.DMA,
            pltpu.SemaphoreType.DMA,
            pltpu.SemaphoreType.DMA,
            pltpu.SemaphoreType.DMA,
        ],
    )(body)


def _pq_kernel(x, w_pq):
    bm = 2000

    def body(x_ref, w_ref, t_ref):
        pq = jnp.dot(x_ref[...], w_ref[...], preferred_element_type=jnp.float32)
        for q in range(4):
            t_ref[2 * q] = pq[:, 128 * q:128 * (q + 1)]
            t_ref[2 * q + 1] = pq[:, 512 + 128 * q:512 + 128 * (q + 1)]

    return pl.pallas_call(
        body,
        grid=(N // bm,),
        in_specs=[pl.BlockSpec((bm, DF), lambda i: (i, 0)),
                  pl.BlockSpec((DF, 2 * H), lambda i: (0, 0))],
        out_specs=pl.BlockSpec((8, bm, 128), lambda i: (0, i, 0)),
        out_shape=jax.ShapeDtypeStruct((8, N, 128), jnp.float32),
    )(x, w_pq)


def _norm_kernel(deg0, deg1):
    bm = 2000

    def body(a_ref, b_ref, cdeg_ref, dinv_ref):
        d = 1.0 + a_ref[:, :16] + b_ref[:, :16]
        cdeg_ref[...] = 1.0 / d
        dinv_ref[...] = lax.rsqrt(d)

    return pl.pallas_call(
        body,
        grid=(N // bm,),
        in_specs=[pl.BlockSpec((bm, 128), lambda i: (i, 0)),
                  pl.BlockSpec((bm, 128), lambda i: (i, 0))],
        out_specs=[pl.BlockSpec((bm, 16), lambda i: (i, 0)),
                   pl.BlockSpec((bm, 16), lambda i: (i, 0))],
        out_shape=[jax.ShapeDtypeStruct((N, 16), jnp.float32),
                   jax.ShapeDtypeStruct((N, 16), jnp.float32)],
    )(deg0, deg1)


def _z1_blk(ps_ref, ea_ref, wc_ref):
    z1 = jnp.concatenate([ps_ref[q] for q in range(4)], axis=1)
    return z1 + jnp.dot(ea_ref[...].astype(jnp.bfloat16),
                        wc_ref[...].astype(jnp.bfloat16),
                        preferred_element_type=jnp.float32)


def _u1_kernel(psum, ea, w1c, dinvw):
    bm = 2000

    def body(ps_ref, ea_ref, wc_ref, dv_ref, *out_refs):
        u = dv_ref[:, :1] * _z1_blk(ps_ref, ea_ref, wc_ref)
        for ch, o_ref in enumerate(out_refs):
            o_ref[...] = u[:, ch * 128:(ch + 1) * 128]

    return pl.pallas_call(
        body,
        grid=(N // bm,),
        in_specs=[pl.BlockSpec((4, bm, 128), lambda i: (0, i, 0)),
                  pl.BlockSpec((bm, 16), lambda i: (i, 0)),
                  pl.BlockSpec((16, H), lambda i: (0, 0)),
                  pl.BlockSpec((bm, 16), lambda i: (i, 0))],
        out_specs=[pl.BlockSpec((bm, 128), lambda i: (i, 0))] * (H // 128),
        out_shape=[jax.ShapeDtypeStruct((N, 128), jnp.float32)] * (H // 128),
    )(psum, ea, w1c, dinvw)


def _layer2_kernel(psum, ea, w1c, agg1, cdegw, dinvw, w2, b1r, b2r):
    bm = 2000
    nhead = N // bm

    def body(ps_ref, ea_ref, wc_ref, agg_ref, cd_ref, dv_ref,
             w_ref, b1_ref, b2_ref, acc_ref, t2_ref, u0_ref, u1_ref):
        i = pl.program_id(0)
        z1 = _z1_blk(ps_ref, ea_ref, wc_ref)
        b1v = b1_ref[...]
        cd = cd_ref[:, :1]
        dv = dv_ref[:, :1]
        agg = agg_ref[0] + agg_ref[1]
        fixed = cd * z1 + dv * agg + b1v
        plain = z1 + b1v
        a = jax.nn.relu(jnp.where(i < nhead, fixed, plain))
        z2 = jnp.dot(a.astype(jnp.bfloat16), w_ref[...].astype(jnp.bfloat16),
                     preferred_element_type=jnp.float32)

        @pl.when(i < nhead)
        def _():
            t2_ref[...] = z2
            u2 = dv * z2
            u0_ref[...] = u2[:, :128]
            u1_ref[...] = u2[:, 128:]

        @pl.when(i == 0)
        def _():
            acc_ref[...] = jnp.zeros_like(acc_ref)

        @pl.when(i >= nhead)
        def _():
            acc_ref[...] += jnp.sum(jax.nn.relu(z2 + b2_ref[...]),
                                    axis=0, keepdims=True)

    head = lambda i: (jnp.minimum(i, nhead - 1), 0)
    return pl.pallas_call(
        body,
        grid=(E // bm,),
        in_specs=[
            pl.BlockSpec((4, bm, 128), lambda i: (0, i, 0)),
            pl.BlockSpec((bm, 16), lambda i: (i, 0)),
            pl.BlockSpec((16, H), lambda i: (0, 0)),
            pl.BlockSpec((NC, bm, H), lambda i: (0, jnp.minimum(i, nhead - 1), 0)),
            pl.BlockSpec((bm, 16), head),
            pl.BlockSpec((bm, 16), head),
            pl.BlockSpec((H, O), lambda i: (0, 0)),
            pl.BlockSpec((1, H), lambda i: (0, 0)),
            pl.BlockSpec((1, O), lambda i: (0, 0)),
        ],
        out_specs=[
            pl.BlockSpec((1, O), lambda i: (0, 0)),
            pl.BlockSpec((bm, O), head),
            pl.BlockSpec((bm, 128), head),
            pl.BlockSpec((bm, 128), head),
        ],
        out_shape=[
            jax.ShapeDtypeStruct((1, O), jnp.float32),
            jax.ShapeDtypeStruct((N, O), jnp.float32),
            jax.ShapeDtypeStruct((N, 128), jnp.float32),
            jax.ShapeDtypeStruct((N, 128), jnp.float32),
        ],
    )(psum, ea, w1c, agg1, cdegw, dinvw, w2, b1r, b2r)


def _final_kernel(partial, t2, agg2, cdegw, dinvw, b2r, fcw_t, fcb_r):
    bm = 2000
    nblk = N // bm

    def body(part_ref, t2_ref, agg_ref, cd_ref, dv_ref, b2_ref,
             fw_ref, fb_ref, out_ref, s_ref):
        i = pl.program_id(0)

        @pl.when(i == 0)
        def _():
            s_ref[...] = part_ref[...]

        rows = jax.nn.relu(cd_ref[:, :1] * t2_ref[...]
                           + dv_ref[:, :1] * (agg_ref[0] + agg_ref[1])
                           + b2_ref[...])
        s_ref[...] += jnp.sum(rows, axis=0, keepdims=True)

        @pl.when(i == nblk - 1)
        def _():
            out_ref[...] = jnp.dot(s_ref[...], fw_ref[...],
                                   preferred_element_type=jnp.float32) + fb_ref[...]

    return pl.pallas_call(
        body,
        grid=(nblk,),
        in_specs=[
            pl.BlockSpec((1, O), lambda i: (0, 0)),
            pl.BlockSpec((bm, O), lambda i: (i, 0)),
            pl.BlockSpec((NC, bm, O), lambda i: (0, i, 0)),
            pl.BlockSpec((bm, 16), lambda i: (i, 0)),
            pl.BlockSpec((bm, 16), lambda i: (i, 0)),
            pl.BlockSpec((1, O), lambda i: (0, 0)),
            pl.BlockSpec((O, O), lambda i: (0, 0)),
            pl.BlockSpec((1, O), lambda i: (0, 0)),
        ],
        out_specs=pl.BlockSpec((1, O), lambda i: (0, 0)),
        out_shape=jax.ShapeDtypeStruct((1, O), jnp.float32),
        scratch_shapes=[pltpu.VMEM((1, O), jnp.float32)],
    )(partial, t2, agg2, cdegw, dinvw, b2r, fcw_t, fcb_r)


def kernel(x, edge_index, edge_attr, W1, b1, W2, b2, fcW, fcb):
    src = edge_index[0]
    dst = edge_index[1]
    src_g = src.reshape(NW, EPT // 40, 40)
    dstN_g = (dst + N).reshape(NW, EPT // 40, 40)
    idx0 = jnp.concatenate([src_g, dstN_g], axis=2)
    src_s = src.reshape(NW, 50, 100)
    dst_s = dst.reshape(NW, 50, 100)

    ones128 = jnp.ones((100, 128), jnp.float32)
    zeros128 = jnp.zeros((128, 128), jnp.float32)

    w_pq = jnp.concatenate([W1[:DF], W1[DF:2 * DF]], axis=1)
    w1c = W1[2 * DF:]
    b1r = b1.reshape(1, H)
    b2r = b2.reshape(1, O)
    fcw_t = fcW.T
    fcb_r = fcb.reshape(1, O)

    degw = _make_deg()(dst_s, ones128, zeros128)
    cdegw, dinvw = _norm_kernel(degw[0], degw[1])
    t_tab = _pq_kernel(x, w_pq).reshape(8 * N, 128)

    psum = _make_asm()(idx0, t_tab)

    u1 = _u1_kernel(psum, edge_attr, w1c, dinvw)
    agg1 = _make_scatter(4)(src_s, dst_s, zeros128, *u1)

    partial, t2, u2c0, u2c1 = _layer2_kernel(
        psum, edge_attr, w1c, agg1, cdegw, dinvw, W2, b1r, b2r)

    agg2 = _make_scatter(2)(src_s, dst_s, zeros128, u2c0, u2c1)

    out = _final_kernel(partial, t2, agg2, cdegw, dinvw, b2r, fcw_t, fcb_r)
    return out.reshape(O)
